# Initial kernel scaffold; baseline (speedup 1.0000x reference)
import jax
import jax.numpy as jnp
from jax.experimental import pallas as pl


def kernel(actions, table):
    raise NotImplementedError("write your pallas kernel here")


# speedup vs baseline: 6.3466x; 6.3466x over previous
import functools

import jax
import jax.numpy as jnp
from jax import lax
from jax.experimental import pallas as pl
from jax.experimental.pallas import tpu as pltpu
from jax.experimental.pallas import tpu_sc as plsc

D = 32
IPG = 128
K = 16
CHUNK = K * IPG


@functools.cache
def _build(total, nc, ns):
    nw = nc * ns
    per_w = total // nw
    rows_per_w = per_w // IPG
    nchunk = per_w // CHUNK

    mesh = plsc.VectorSubcoreMesh(
        core_axis_name="c", subcore_axis_name="s",
        num_cores=nc, num_subcores=ns)

    @functools.partial(
        pl.kernel,
        out_type=jax.ShapeDtypeStruct((total, D), jnp.float32),
        mesh=mesh,
        scratch_types=[
            pltpu.VMEM((K, IPG), jnp.int32),
            pltpu.VMEM((CHUNK, D), jnp.float32),
            pltpu.SemaphoreType.DMA,
        ],
        compiler_params=pltpu.CompilerParams(use_tc_tiling_on_sc=False),
    )
    def gather_kernel(table_hbm, idx_hbm, out_hbm, idx_v, rows_v, sem):
        wid = lax.axis_index("s") * nc + lax.axis_index("c")
        row0 = wid * rows_per_w
        base = wid * per_w

        def chunk_body(c, carry):
            pltpu.sync_copy(idx_hbm.at[pl.ds(row0 + c * K, K)], idx_v)
            for j in range(K):
                pltpu.make_async_copy(
                    table_hbm.at[idx_v.at[j]],
                    rows_v.at[pl.ds(j * IPG, IPG)],
                    sem,
                ).start()
            for j in range(K):
                pltpu.make_async_copy(
                    table_hbm.at[idx_v.at[j]],
                    rows_v.at[pl.ds(j * IPG, IPG)],
                    sem,
                ).wait()
            pltpu.sync_copy(rows_v, out_hbm.at[pl.ds(base + c * CHUNK, CHUNK)])
            return carry

        lax.fori_loop(0, nchunk, chunk_body, 0)

    return gather_kernel


def kernel(actions, table):
    b, l = actions.shape
    total = b * l
    info = plsc.get_sparse_core_info()
    nc, ns = info.num_cores, info.num_subcores
    idx2d = actions.astype(jnp.int32).reshape(total // IPG, IPG)
    out = _build(total, nc, ns)(table, idx2d)
    return out.reshape(b, l, 1, D)


# speedup vs baseline: 6.5045x; 1.0249x over previous
import functools

import jax
import jax.numpy as jnp
from jax import lax
from jax.experimental import pallas as pl
from jax.experimental.pallas import tpu as pltpu
from jax.experimental.pallas import tpu_sc as plsc

D = 32
IPG = 128
K = 10
CHUNK = K * IPG


@functools.cache
def _build(total, nc, ns):
    nw = nc * ns
    per_w = total // nw
    rows_per_w = per_w // IPG
    nch = per_w // CHUNK
    assert nch % 2 == 0 and nch >= 4 and nch * CHUNK == per_w

    mesh = plsc.VectorSubcoreMesh(
        core_axis_name="c", subcore_axis_name="s",
        num_cores=nc, num_subcores=ns)

    @functools.partial(
        pl.kernel,
        out_type=jax.ShapeDtypeStruct((total, D), jnp.float32),
        mesh=mesh,
        scratch_types=[
            pltpu.VMEM((2, K, IPG), jnp.int32),
            pltpu.VMEM((2, CHUNK, D), jnp.float32),
            pltpu.SemaphoreType.DMA,
            pltpu.SemaphoreType.DMA,
            pltpu.SemaphoreType.DMA,
            pltpu.SemaphoreType.DMA,
            pltpu.SemaphoreType.DMA,
            pltpu.SemaphoreType.DMA,
        ],
        compiler_params=pltpu.CompilerParams(use_tc_tiling_on_sc=False),
    )
    def gather_kernel(table_hbm, idx_hbm, out_hbm, idx_v, rows_v,
                      isem0, isem1, gsem0, gsem1, osem0, osem1):
        wid = lax.axis_index("s") * nc + lax.axis_index("c")
        row0 = wid * rows_per_w
        base = wid * per_w
        isem = (isem0, isem1)
        gsem = (gsem0, gsem1)
        osem = (osem0, osem1)

        def s_idx(c, b):
            pltpu.make_async_copy(
                idx_hbm.at[pl.ds(row0 + c * K, K)], idx_v.at[b], isem[b]
            ).start()

        def w_idx(b):
            pltpu.make_async_copy(
                idx_hbm.at[pl.ds(row0, K)], idx_v.at[b], isem[b]
            ).wait()

        def g_fire(b):
            for j in range(K):
                pltpu.make_async_copy(
                    table_hbm.at[idx_v.at[b, j]],
                    rows_v.at[b, pl.ds(j * IPG, IPG)],
                    gsem[b],
                ).start()

        def g_drain(b):
            for j in range(K):
                pltpu.make_async_copy(
                    table_hbm.at[idx_v.at[b, j]],
                    rows_v.at[b, pl.ds(j * IPG, IPG)],
                    gsem[b],
                ).wait()

        def s_out(c, b):
            pltpu.make_async_copy(
                rows_v.at[b], out_hbm.at[pl.ds(base + c * CHUNK, CHUNK)],
                osem[b],
            ).start()

        def w_out(b):
            pltpu.make_async_copy(
                rows_v.at[b], out_hbm.at[pl.ds(base, CHUNK)], osem[b]
            ).wait()

        s_idx(0, 0)
        s_idx(1, 1)
        w_idx(0)
        g_fire(0)
        w_idx(1)
        g_fire(1)
        g_drain(0)
        s_out(0, 0)
        s_idx(2, 0)

        def pair(p, carry):
            cA = 2 * p + 1
            w_idx(0)
            w_out(0)
            g_fire(0)
            g_drain(1)
            s_out(cA, 1)
            @pl.when(cA + 2 < nch)
            def _():
                s_idx(cA + 2, 1)
            cB = cA + 1
            w_idx(1)
            w_out(1)
            g_fire(1)
            g_drain(0)
            s_out(cB, 0)
            @pl.when(cB + 2 < nch)
            def _():
                s_idx(cB + 2, 0)
            return carry

        lax.fori_loop(0, (nch - 2) // 2, pair, 0)

        g_drain(1)
        s_out(nch - 1, 1)
        w_out(0)
        w_out(1)

    return gather_kernel


def kernel(actions, table):
    b, l = actions.shape
    total = b * l
    info = plsc.get_sparse_core_info()
    nc, ns = info.num_cores, info.num_subcores
    idx2d = actions.astype(jnp.int32).reshape(total // IPG, IPG)
    out = _build(total, nc, ns)(table, idx2d)
    return out.reshape(b, l, 1, D)
